# baseline (speedup 1.0000x reference)
import functools
import math

import numpy as np
import jax
import jax.numpy as jnp
from jax import lax
from jax.experimental import pallas as pl
from jax.experimental.pallas import tpu as pltpu
from jax.experimental.pallas import tpu_sc as plsc

_VOCAB = 100000
_D = 1024
_B = 4
_S = 4096
_NC = 2
_NS = 16
_NW = _NC * _NS
_PPW = _S // _NW
_CHUNK = 32
_NCHUNK = _PPW // _CHUNK
_SCALE = math.sqrt(_D)
_LANES = 16
_VPR = _D // _LANES


def _pos_encoding() -> np.ndarray:
    pos = np.arange(_S, dtype=np.float32)[:, None]
    i = np.arange(_D // 2, dtype=np.float32)[None, :]
    angle = pos / np.power(10000.0, (2.0 * i) / _D)
    pe = np.zeros((_S, _D), dtype=np.float32)
    pe[:, 0::2] = np.sin(angle)
    pe[:, 1::2] = np.cos(angle)
    return pe


_PE = _pos_encoding()


def _body(table, seqs, pe, out, idx_v, pe_v, buf, gsem):
    wid = lax.axis_index("s") * _NC + lax.axis_index("c")
    pos0 = wid * _PPW

    for b in range(_B):
        pltpu.sync_copy(seqs.at[b, pl.ds(pos0, _PPW)], idx_v.at[b])

    for i in range(_NCHUNK):
        pltpu.sync_copy(pe.at[pl.ds(pos0 + i * _CHUNK, _CHUNK)], pe_v)
        for b in range(_B):
            pltpu.async_copy(
                table.at[idx_v.at[b, pl.ds(i * _CHUNK, _CHUNK)]], buf, gsem
            ).wait()

            @pl.loop(0, _CHUNK)
            def _rows(r):
                @pl.loop(0, _VPR, unroll=8)
                def _vecs(j):
                    sl = pl.ds(j * _LANES, _LANES)
                    buf[r, sl] = buf[r, sl] * _SCALE + pe_v[r, sl]

            pltpu.sync_copy(buf, out.at[b, pl.ds(pos0 + i * _CHUNK, _CHUNK)])


@jax.jit
def _embed(seqs, table, pe):
    k = pl.kernel(
        _body,
        out_type=jax.ShapeDtypeStruct((_B, _S, _D), jnp.float32),
        mesh=plsc.VectorSubcoreMesh(core_axis_name="c", subcore_axis_name="s"),
        scratch_types=[
            pltpu.VMEM((_B, _PPW), jnp.int32),
            pltpu.VMEM((_CHUNK, _D), jnp.float32),
            pltpu.VMEM((_CHUNK, _D), jnp.float32),
            pltpu.SemaphoreType.DMA,
        ],
    )
    return k(table, seqs, pe)


def kernel(seqs, embed_weight):
    pe = jnp.asarray(_PE)
    return _embed(seqs, embed_weight, pe)


# speedup vs baseline: 1.1518x; 1.1518x over previous
import functools
import math

import numpy as np
import jax
import jax.numpy as jnp
from jax import lax
from jax.experimental import pallas as pl
from jax.experimental.pallas import tpu as pltpu
from jax.experimental.pallas import tpu_sc as plsc

_VOCAB = 100000
_D = 1024
_B = 4
_S = 4096
_NC = 2
_NS = 16
_NW = _NC * _NS
_PPW = _S // _NW
_CHUNK = 32
_NCHUNK = _PPW // _CHUNK
_SCALE = math.sqrt(_D)
_LANES = 16
_VPR = _D // _LANES


def _pos_encoding() -> np.ndarray:
    pos = np.arange(_S, dtype=np.float32)[:, None]
    i = np.arange(_D // 2, dtype=np.float32)[None, :]
    angle = pos / np.power(10000.0, (2.0 * i) / _D)
    pe = np.zeros((_S, _D), dtype=np.float32)
    pe[:, 0::2] = np.sin(angle)
    pe[:, 1::2] = np.cos(angle)
    return pe


_PE = _pos_encoding()


def _compute(buf, pe_v):
    @pl.loop(0, _CHUNK)
    def _rows(r):
        @pl.loop(0, _VPR, unroll=8)
        def _vecs(j):
            sl = pl.ds(j * _LANES, _LANES)
            buf[r, sl] = buf[r, sl] * _SCALE + pe_v[r, sl]


def _body(table, seqs, pe, out, idx_v, pe_v, buf0, buf1,
          gsem0, gsem1, ssem0, ssem1, psem):
    wid = lax.axis_index("s") * _NC + lax.axis_index("c")
    pos0 = wid * _PPW
    bufs = (buf0, buf1)
    gsems = (gsem0, gsem1)
    ssems = (ssem0, ssem1)

    for b in range(_B):
        pltpu.sync_copy(seqs.at[b, pl.ds(pos0, _PPW)], idx_v.at[b])

    def start_gather(u, p):
        i, b = divmod(u, _B)
        return pltpu.async_copy(
            table.at[idx_v.at[b, pl.ds(i * _CHUNK, _CHUNK)]], bufs[p], gsems[p]
        )

    nunits = _NCHUNK * _B
    pe_desc = pltpu.async_copy(pe.at[pl.ds(pos0, _CHUNK)], pe_v, psem)
    g = {0: start_gather(0, 0)}
    s = {}
    for u in range(nunits):
        p = u & 1
        i, b = divmod(u, _B)
        if u + 1 < nunits:
            if u >= 1:
                s[1 - p].wait()
            g[1 - p] = start_gather(u + 1, 1 - p)
        if b == 0:
            pe_desc.wait()
        g[p].wait()
        _compute(bufs[p], pe_v)
        if b == _B - 1 and i + 1 < _NCHUNK:
            pe_desc = pltpu.async_copy(
                pe.at[pl.ds(pos0 + (i + 1) * _CHUNK, _CHUNK)], pe_v, psem)
        s[p] = pltpu.async_copy(
            bufs[p], out.at[b, pl.ds(pos0 + i * _CHUNK, _CHUNK)], ssems[p])
    s[0].wait()
    s[1].wait()


@jax.jit
def _embed(seqs, table, pe):
    k = pl.kernel(
        _body,
        out_type=jax.ShapeDtypeStruct((_B, _S, _D), jnp.float32),
        mesh=plsc.VectorSubcoreMesh(core_axis_name="c", subcore_axis_name="s"),
        scratch_types=[
            pltpu.VMEM((_B, _PPW), jnp.int32),
            pltpu.VMEM((_CHUNK, _D), jnp.float32),
            pltpu.VMEM((_CHUNK, _D), jnp.float32),
            pltpu.VMEM((_CHUNK, _D), jnp.float32),
            pltpu.SemaphoreType.DMA,
            pltpu.SemaphoreType.DMA,
            pltpu.SemaphoreType.DMA,
            pltpu.SemaphoreType.DMA,
            pltpu.SemaphoreType.DMA,
        ],
    )
    return k(table, seqs, pe)


def kernel(seqs, embed_weight):
    pe = jnp.asarray(_PE)
    return _embed(seqs, embed_weight, pe)


# speedup vs baseline: 2.8347x; 2.4611x over previous
import math

import numpy as np
import jax
import jax.numpy as jnp
from jax import lax
from jax.experimental import pallas as pl
from jax.experimental.pallas import tpu as pltpu
from jax.experimental.pallas import tpu_sc as plsc

_D = 1024
_B = 4
_S = 4096
_NC = 2
_NS = 16
_NW = _NC * _NS
_PPW = _S // _NW
_CH = 8
_NCHUNK = _PPW // _CH
_SCALE = math.sqrt(_D)
_LANES = 16
_VPR = _D // _LANES
_NGRP = 3


def _pos_encoding() -> np.ndarray:
    pos = np.arange(_S, dtype=np.float32)[:, None]
    i = np.arange(_D // 2, dtype=np.float32)[None, :]
    angle = pos / np.power(10000.0, (2.0 * i) / _D)
    pe = np.zeros((_S, _D), dtype=np.float32)
    pe[:, 0::2] = np.sin(angle)
    pe[:, 1::2] = np.cos(angle)
    return pe


_PE = _pos_encoding()


def _compute(bufs, pe_v):
    @pl.loop(0, _CH)
    def _rows(r):
        @plsc.parallel_loop(0, _VPR, unroll=4)
        def _vecs(j):
            sl = pl.ds(j * _LANES, _LANES)
            p = pe_v[r, sl]
            for buf in bufs:
                buf[r, sl] = buf[r, sl] * _SCALE + p


def _body(table, seqs, pe, out, idx_v, pe0, pe1, *rest):
    grp = [[rest[g * _B + b] for b in range(_B)] for g in range(_NGRP)]
    gsems = rest[_NGRP * _B:_NGRP * _B + _NGRP]
    ssems = rest[_NGRP * _B + _NGRP:_NGRP * _B + 2 * _NGRP]
    psems = rest[_NGRP * _B + 2 * _NGRP:]
    pes = (pe0, pe1)

    wid = lax.axis_index("s") * _NC + lax.axis_index("c")
    pos0 = wid * _PPW

    for b in range(_B):
        pltpu.sync_copy(seqs.at[b, pl.ds(pos0, _PPW)], idx_v.at[b])

    def start_gathers(i):
        g = i % _NGRP
        return [
            pltpu.async_copy(
                table.at[idx_v.at[b, pl.ds(i * _CH, _CH)]], grp[g][b], gsems[g])
            for b in range(_B)
        ]

    def start_pe(i):
        return pltpu.async_copy(
            pe.at[pl.ds(pos0 + i * _CH, _CH)], pes[i & 1], psems[i & 1])

    def start_stores(i):
        g = i % _NGRP
        return [
            pltpu.async_copy(
                grp[g][b], out.at[b, pl.ds(pos0 + i * _CH, _CH)], ssems[g])
            for b in range(_B)
        ]

    gd = {0: start_gathers(0)}
    pd = {0: start_pe(0)}
    sd = {}
    for i in range(_NCHUNK):
        g = i % _NGRP
        if i + 1 < _NCHUNK:
            if i + 1 >= _NGRP:
                for d in sd.pop(i + 1 - _NGRP):
                    d.wait()
            gd[i + 1] = start_gathers(i + 1)
            pd[i + 1] = start_pe(i + 1)
        pd.pop(i).wait()
        for d in gd.pop(i):
            d.wait()
        _compute(grp[g], pes[i & 1])
        sd[i] = start_stores(i)
    for i in sorted(sd):
        for d in sd[i]:
            d.wait()


def _embed(seqs, table, pe):
    k = pl.kernel(
        _body,
        out_type=jax.ShapeDtypeStruct((_B, _S, _D), jnp.float32),
        mesh=plsc.VectorSubcoreMesh(core_axis_name="c", subcore_axis_name="s"),
        scratch_types=[
            pltpu.VMEM((_B, _PPW), jnp.int32),
            pltpu.VMEM((_CH, _D), jnp.float32),
            pltpu.VMEM((_CH, _D), jnp.float32),
        ]
        + [pltpu.VMEM((_CH, _D), jnp.float32) for _ in range(_NGRP * _B)]
        + [pltpu.SemaphoreType.DMA for _ in range(2 * _NGRP + 2)],
    )
    return k(table, seqs, pe)


def kernel(seqs, embed_weight):
    pe = jnp.asarray(_PE)
    return jax.jit(_embed)(seqs, embed_weight, pe)
